# Initial kernel scaffold; baseline (speedup 1.0000x reference)
import jax
import jax.numpy as jnp
from jax.experimental import pallas as pl


def kernel(names, x, y, z, categorical, numerical, atom_table, num_table):
    raise NotImplementedError("write your pallas kernel here")


# speedup vs baseline: 1.7290x; 1.7290x over previous
import math
import jax
import jax.numpy as jnp
from jax.experimental import pallas as pl
from jax.experimental.pallas import tpu as pltpu

DIM = 40
HALF = DIM // 2
N_ROWS = 16384
BLOCK = 1024


def _embed_block(names_ref, num_ref, x_ref, y_ref, z_ref, cat_ref,
                 at_ref, nt_ref, freqs_ref, out_ref):
    b = names_ref.shape[0]
    col = jax.lax.broadcasted_iota(jnp.int32, (1, DIM), 1)
    oh_a = (names_ref[...] == col).astype(jnp.float32)
    oh_n = (num_ref[...] == col).astype(jnp.float32)
    atoms = jnp.dot(oh_a, at_ref[...], preferred_element_type=jnp.float32)
    nums = jnp.dot(oh_n, nt_ref[...], preferred_element_type=jnp.float32)

    freqs = freqs_ref[...]
    ax = x_ref[...] * freqs
    ay = y_ref[...] * freqs
    az = z_ref[...] * freqs

    out_ref[...] = jnp.concatenate(
        [atoms,
         jnp.sin(ax), jnp.cos(ax),
         jnp.sin(ay), jnp.cos(ay),
         jnp.sin(az), jnp.cos(az),
         cat_ref[...], nums], axis=1)


def kernel(names, x, y, z, categorical, numerical, atom_table, num_table):
    n = names.shape[0]
    grid = (n // BLOCK,)
    exponent = 2.0 * jnp.arange(HALF, dtype=jnp.float32) / DIM
    freqs = (2.0 * math.pi / (10000.0 ** exponent)).reshape(1, HALF)

    row_spec = lambda w: pl.BlockSpec((BLOCK, w), lambda i: (i, 0))
    table_spec = pl.BlockSpec((DIM, DIM), lambda i: (0, 0))

    return pl.pallas_call(
        _embed_block,
        grid=grid,
        in_specs=[
            row_spec(1),
            row_spec(1),
            row_spec(1),
            row_spec(1),
            row_spec(1),
            row_spec(DIM),
            table_spec,
            table_spec,
            pl.BlockSpec((1, HALF), lambda i: (0, 0)),
        ],
        out_specs=row_spec(6 * DIM),
        out_shape=jax.ShapeDtypeStruct((n, 6 * DIM), jnp.float32),
        compiler_params=pltpu.CompilerParams(
            dimension_semantics=("arbitrary",)),
    )(names.reshape(n, 1), numerical.reshape(n, 1), x, y, z,
      categorical, atom_table, num_table, freqs)


# speedup vs baseline: 4.1137x; 2.3792x over previous
import math
import jax
import jax.numpy as jnp
import numpy as np
from jax.experimental import pallas as pl
from jax.experimental.pallas import tpu as pltpu

DIM = 40
HALF = DIM // 2
BLOCK = 1024


def _sin_turns(u):
    u = u - jnp.floor(u)
    t = 4.0 * u
    q = jnp.floor(t + 0.5)
    theta = (t - q) * (math.pi / 2.0)
    th2 = theta * theta
    s = -1.0 / 5040.0
    s = s * th2 + 1.0 / 120.0
    s = s * th2 - 1.0 / 6.0
    s = s * th2 + 1.0
    s = s * theta
    c = -1.0 / 720.0
    c = c * th2 + 1.0 / 24.0
    c = c * th2 - 1.0 / 2.0
    c = c * th2 + 1.0
    qm = q.astype(jnp.int32) & 3
    mag = jnp.where((qm & 1) == 1, c, s)
    return jnp.where(qm >= 2, -mag, mag)


def _embed_block(names_ref, num_ref, xyz_ref, cat_ref,
                 at_ref, nt_ref, fm_ref, off_ref, out_ref):
    col = jax.lax.broadcasted_iota(jnp.int32, (1, DIM), 1)
    oh_a = (names_ref[...] == col).astype(jnp.float32)
    oh_n = (num_ref[...] == col).astype(jnp.float32)
    atoms = jnp.dot(oh_a, at_ref[...], preferred_element_type=jnp.float32)
    nums = jnp.dot(oh_n, nt_ref[...], preferred_element_type=jnp.float32)

    u = jnp.dot(xyz_ref[...], fm_ref[...],
                preferred_element_type=jnp.float32) + off_ref[...]
    trig = _sin_turns(u)

    out_ref[:, 0:DIM] = atoms
    out_ref[:, DIM:4 * DIM] = trig
    out_ref[:, 4 * DIM:5 * DIM] = cat_ref[...]
    out_ref[:, 5 * DIM:6 * DIM] = nums


def kernel(names, x, y, z, categorical, numerical, atom_table, num_table):
    n = names.shape[0]
    grid = (n // BLOCK,)

    inv = (10000.0 ** (-2.0 * np.arange(HALF) / DIM)).astype(np.float32)
    fm = np.zeros((3, 3 * DIM), dtype=np.float32)
    off = np.zeros((1, 3 * DIM), dtype=np.float32)
    for j in range(3):
        fm[j, j * DIM:j * DIM + HALF] = inv
        fm[j, j * DIM + HALF:(j + 1) * DIM] = inv
        off[0, j * DIM + HALF:(j + 1) * DIM] = 0.25
    fm = jnp.asarray(fm)
    off = jnp.asarray(off)
    xyz = jnp.concatenate([x, y, z], axis=1)

    row_spec = lambda w: pl.BlockSpec((BLOCK, w), lambda i: (i, 0))
    fix_spec = lambda h, w: pl.BlockSpec((h, w), lambda i: (0, 0))

    return pl.pallas_call(
        _embed_block,
        grid=grid,
        in_specs=[
            row_spec(1),
            row_spec(1),
            row_spec(3),
            row_spec(DIM),
            fix_spec(DIM, DIM),
            fix_spec(DIM, DIM),
            fix_spec(3, 3 * DIM),
            fix_spec(1, 3 * DIM),
        ],
        out_specs=row_spec(6 * DIM),
        out_shape=jax.ShapeDtypeStruct((n, 6 * DIM), jnp.float32),
        compiler_params=pltpu.CompilerParams(
            dimension_semantics=("arbitrary",)),
    )(names.reshape(n, 1), numerical.reshape(n, 1), xyz,
      categorical, atom_table, num_table, fm, off)
